# baseline (speedup 1.0000x reference)
import jax
import jax.numpy as jnp
from jax.experimental import pallas as pl
from jax.experimental.pallas import tpu as pltpu

NUM_POINTS = 2_000_000
N_NEW = 1_000_000
N_FEAT = 1000
D_FEAT = 512

TILE = 128
P_LO = (N_NEW // TILE) * TILE
P_HI = P_LO + TILE

CHUNK = 8192
RING = 8


def _chunks(base, total):
    out = []
    off = base
    while off < base + total:
        sz = min(CHUNK, base + total - off)
        out.append((off, sz))
        off += sz
    return out


def _copy_body(np_ref, bnd_ref, lbnd_ref, pose_ref, nl_ref, feat_ref,
               pbuf_ref, lbuf_ref, out_p, out_pose, out_l, out_f,
               b0, b1, b2, b3, b4, b5, b6, b7, rsem, wsem, dsem):
    bufs = (b0, b1, b2, b3, b4, b5, b6, b7)

    direct = [
        pltpu.make_async_copy(nl_ref.at[pl.ds(0, P_LO)],
                              out_l.at[pl.ds(0, P_LO)], dsem),
        pltpu.make_async_copy(lbnd_ref, out_l.at[pl.ds(P_LO, TILE)], dsem),
        pltpu.make_async_copy(lbuf_ref.at[pl.ds(P_HI, NUM_POINTS - P_HI)],
                              out_l.at[pl.ds(P_HI, NUM_POINTS - P_HI)], dsem),
        pltpu.make_async_copy(bnd_ref, out_p.at[pl.ds(P_LO, TILE)], dsem),
        pltpu.make_async_copy(feat_ref, out_f, dsem),
        pltpu.make_async_copy(pose_ref, out_pose, dsem),
    ]
    for c in direct:
        c.start()

    np_jobs = [(np_ref, off, sz) for off, sz in _chunks(0, P_LO)]
    pb_jobs = [(pbuf_ref, off, sz)
               for off, sz in _chunks(P_HI, NUM_POINTS - P_HI)]
    jobs = []
    for a, b in zip(np_jobs, pb_jobs):
        jobs.append(a)
        jobs.append(b)
    jobs += np_jobs[len(pb_jobs):] + pb_jobs[len(np_jobs):]
    n = len(jobs)
    reads, writes = [], []
    for k, (src, off, sz) in enumerate(jobs):
        buf = bufs[k % RING]
        vslice = buf.at[pl.ds(0, sz)] if sz < CHUNK else buf
        reads.append(pltpu.make_async_copy(
            src.at[pl.ds(off, sz)], vslice, rsem))
        writes.append(pltpu.make_async_copy(
            vslice, out_p.at[pl.ds(off, sz)], wsem))
    for k in range(n + 1):
        if k < n:
            if k >= RING:
                writes[k - RING].wait()
            reads[k].start()
        if k >= 1:
            reads[k - 1].wait()
            writes[k - 1].start()
    for k in range(max(0, n - RING), n):
        writes[k].wait()

    for c in direct:
        c.wait()


def kernel(new_points, pose, new_point_label, new_label_feature,
           points_buf, points_label_buf, label_feature_buf, pose_buf):
    del label_feature_buf, pose_buf
    boundary = jnp.concatenate(
        [new_points[P_LO:], points_buf[N_NEW:P_HI]], axis=0)
    lboundary = jnp.concatenate(
        [new_point_label[P_LO:], points_label_buf[N_NEW:P_HI]], axis=0)
    out_p, out_pose, out_l, out_f = pl.pallas_call(
        _copy_body,
        in_specs=[pl.BlockSpec(memory_space=pl.ANY)] * 8,
        out_specs=[pl.BlockSpec(memory_space=pl.ANY)] * 4,
        out_shape=(
            jax.ShapeDtypeStruct((NUM_POINTS, 3), jnp.float32),
            jax.ShapeDtypeStruct((4, 4), jnp.float32),
            jax.ShapeDtypeStruct((NUM_POINTS,), jnp.int32),
            jax.ShapeDtypeStruct((N_FEAT, D_FEAT), jnp.float32),
        ),
        scratch_shapes=(
            [pltpu.VMEM((CHUNK, 3), jnp.float32)] * 8
            + [pltpu.SemaphoreType.DMA] * 3
        ),
    )(new_points, boundary, lboundary, pose, new_point_label,
      new_label_feature, points_buf, points_label_buf)
    return out_p, out_pose, out_l, out_f


# speedup vs baseline: 1.2013x; 1.2013x over previous
import jax
import jax.numpy as jnp
from jax.experimental import pallas as pl
from jax.experimental.pallas import tpu as pltpu

NUM_POINTS = 2_000_000
N_NEW = 1_000_000
N_FEAT = 1000
D_FEAT = 512

TILE = 128
P_LO = (N_NEW // TILE) * TILE
P_HI = P_LO + TILE

CHUNK = 8192
RING = 4


def _chunks(base, total):
    out = []
    off = base
    while off < base + total:
        sz = min(CHUNK, base + total - off)
        out.append((off, sz))
        off += sz
    return out


def _copy_body(np_ref, bnd_ref, lbnd_ref, pose_ref, nl_ref, feat_ref,
               pbuf_ref, lbuf_ref, out_p, out_pose, out_l, out_f,
               b0, b1, b2, b3, rsem, wsem, dsem):
    bufs = (b0, b1, b2, b3)

    del pbuf_ref

    direct = [
        pltpu.make_async_copy(nl_ref.at[pl.ds(0, P_LO)],
                              out_l.at[pl.ds(0, P_LO)], dsem),
        pltpu.make_async_copy(lbnd_ref, out_l.at[pl.ds(P_LO, TILE)], dsem),
        pltpu.make_async_copy(lbuf_ref.at[pl.ds(P_HI, NUM_POINTS - P_HI)],
                              out_l.at[pl.ds(P_HI, NUM_POINTS - P_HI)], dsem),
        pltpu.make_async_copy(bnd_ref, out_p.at[pl.ds(P_LO, TILE)], dsem),
        pltpu.make_async_copy(feat_ref, out_f, dsem),
        pltpu.make_async_copy(pose_ref, out_pose, dsem),
    ]
    for c in direct:
        c.start()

    jobs = [(np_ref, off, sz) for off, sz in _chunks(0, P_LO)]
    n = len(jobs)
    reads, writes = [], []
    for k, (src, off, sz) in enumerate(jobs):
        buf = bufs[k % RING]
        vslice = buf.at[pl.ds(0, sz)] if sz < CHUNK else buf
        reads.append(pltpu.make_async_copy(
            src.at[pl.ds(off, sz)], vslice, rsem))
        writes.append(pltpu.make_async_copy(
            vslice, out_p.at[pl.ds(off, sz)], wsem))
    for k in range(n + 1):
        if k < n:
            if k >= RING:
                writes[k - RING].wait()
            reads[k].start()
        if k >= 1:
            reads[k - 1].wait()
            writes[k - 1].start()
    for k in range(max(0, n - RING), n):
        writes[k].wait()

    for c in direct:
        c.wait()


def kernel(new_points, pose, new_point_label, new_label_feature,
           points_buf, points_label_buf, label_feature_buf, pose_buf):
    del label_feature_buf, pose_buf
    boundary = jnp.concatenate(
        [new_points[P_LO:], points_buf[N_NEW:P_HI]], axis=0)
    lboundary = jnp.concatenate(
        [new_point_label[P_LO:], points_label_buf[N_NEW:P_HI]], axis=0)
    out_p, out_pose, out_l, out_f = pl.pallas_call(
        _copy_body,
        in_specs=[pl.BlockSpec(memory_space=pl.ANY)] * 8,
        out_specs=[pl.BlockSpec(memory_space=pl.ANY)] * 4,
        out_shape=(
            jax.ShapeDtypeStruct((NUM_POINTS, 3), jnp.float32),
            jax.ShapeDtypeStruct((4, 4), jnp.float32),
            jax.ShapeDtypeStruct((NUM_POINTS,), jnp.int32),
            jax.ShapeDtypeStruct((N_FEAT, D_FEAT), jnp.float32),
        ),
        scratch_shapes=[
            pltpu.VMEM((CHUNK, 3), jnp.float32),
            pltpu.VMEM((CHUNK, 3), jnp.float32),
            pltpu.VMEM((CHUNK, 3), jnp.float32),
            pltpu.VMEM((CHUNK, 3), jnp.float32),
            pltpu.SemaphoreType.DMA,
            pltpu.SemaphoreType.DMA,
            pltpu.SemaphoreType.DMA,
        ],
        input_output_aliases={6: 0},
    )(new_points, boundary, lboundary, pose, new_point_label,
      new_label_feature, points_buf, points_label_buf)
    return out_p, out_pose, out_l, out_f
